# Initial kernel scaffold; baseline (speedup 1.0000x reference)
import jax
import jax.numpy as jnp
from jax.experimental import pallas as pl


def kernel(z, codebook):
    raise NotImplementedError("write your pallas kernel here")


# speedup vs baseline: 9.0888x; 9.0888x over previous
import functools

import jax
import jax.numpy as jnp
from jax import lax
from jax.experimental import pallas as pl
from jax.experimental.pallas import tpu as pltpu
from jax.experimental.pallas import tpu_sc as plsc

CB = 8192
D = 64
N_ROWS = 9216
BLK = 256
GRID = N_ROWS // BLK
NG = 4
GW = CB // NG

NC = 2
NS = 16
NW = NC * NS
BPW = N_ROWS // NW
CH = 96
NCH = BPW // CH

_COMMIT_WEIGHT = 0.25
_QUANT_WEIGHT = 1.0


def _dist_argmin_body(z_ref, cb_ref, idx_ref, dsum_ref):
    step = pl.program_id(0)
    z = z_ref[...]
    c = cb_ref[...]
    sz = jnp.sum(z * z, axis=1, keepdims=True)
    sc = jnp.sum(c * c, axis=1)
    z2 = 2.0 * z
    mm2 = lax.dot_general(
        z2, c,
        dimension_numbers=(((1,), (1,)), ((), ())),
        preferred_element_type=jnp.float32,
    )
    d = (sz + sc[None, :]) - mm2
    vmin = None
    for g in range(NG):
        dg = d[:, g * GW:(g + 1) * GW]
        mg = jnp.min(dg, axis=1, keepdims=True)
        hit = dg == mg
        iota = lax.broadcasted_iota(jnp.int32, dg.shape, 1) + g * GW
        ig = jnp.min(jnp.where(hit, iota, CB), axis=1, keepdims=True)
        bg = mg.astype(jnp.bfloat16).astype(jnp.float32)
        if g == 0:
            vmin, idx, carry_b = mg, ig, bg
        else:
            take = mg < carry_b
            idx = jnp.where(take, ig, idx)
            carry_b = jnp.where(take, bg, carry_b)
            vmin = jnp.minimum(vmin, mg)
    idx_ref[...] = idx[:, 0]
    dmin = vmin

    @pl.when(step == 0)
    def _init():
        dsum_ref[0, 0] = 0.0

    dsum_ref[0, 0] += jnp.sum(dmin)


def _tc_dist_argmin(flat_z, codebook):
    return pl.pallas_call(
        _dist_argmin_body,
        grid=(GRID,),
        in_specs=[
            pl.BlockSpec((BLK, D), lambda i: (i, 0)),
            pl.BlockSpec((CB, D), lambda i: (0, 0)),
        ],
        out_specs=[
            pl.BlockSpec((BLK,), lambda i: (i,)),
            pl.BlockSpec(memory_space=pltpu.SMEM),
        ],
        out_shape=[
            jax.ShapeDtypeStruct((N_ROWS,), jnp.int32),
            jax.ShapeDtypeStruct((1, 1), jnp.float32),
        ],
    )(flat_z, codebook)


def _sc_gather_body(cb_hbm, idx_hbm, out_hbm, idx_v, rows_v, sem):
    wid = lax.axis_index("s") * NC + lax.axis_index("c")
    pltpu.sync_copy(idx_hbm.at[wid], idx_v)
    copies = [
        pltpu.async_copy(
            cb_hbm.at[idx_v.at[j]], rows_v.at[pl.ds(j * CH, CH)], sem
        )
        for j in range(NCH)
    ]
    for cp in copies:
        cp.wait()
    pltpu.sync_copy(rows_v, out_hbm.at[pl.ds(wid * BPW, BPW)])


@functools.cache
def _sc_gather():
    return pl.kernel(
        _sc_gather_body,
        mesh=plsc.VectorSubcoreMesh(core_axis_name="c", subcore_axis_name="s"),
        out_type=jax.ShapeDtypeStruct((N_ROWS, D), jnp.float32),
        scratch_types=[
            pltpu.VMEM((NCH, CH), jnp.int32),
            pltpu.VMEM((BPW, D), jnp.float32),
            pltpu.SemaphoreType.DMA,
        ],
        compiler_params=pltpu.CompilerParams(use_tc_tiling_on_sc=False),
    )


def kernel(z, codebook):
    b, l, h = z.shape
    flat_z = z.reshape(b * l, h)
    idx_flat, dsum = _tc_dist_argmin(flat_z, codebook)
    q = _sc_gather()(codebook, idx_flat.reshape(NW, NCH, CH))
    mse = dsum[0, 0] / jnp.float32(b * l * h)
    loss = _COMMIT_WEIGHT * mse + _QUANT_WEIGHT * mse
    return (
        q.reshape(b, l, h),
        idx_flat.reshape(b, l),
        loss,
        mse,
        mse,
    )


# speedup vs baseline: 10.6526x; 1.1721x over previous
import functools

import jax
import jax.numpy as jnp
from jax import lax
from jax.experimental import pallas as pl
from jax.experimental.pallas import tpu as pltpu
from jax.experimental.pallas import tpu_sc as plsc

CB = 8192
D = 64
N_ROWS = 9216
BLK = 256
GRID = N_ROWS // BLK
NG = 4
GW = CB // NG

NC = 2
NS = 16
NW = NC * NS
BPW = N_ROWS // NW
CH = 96
NCH = BPW // CH

_COMMIT_WEIGHT = 0.25
_QUANT_WEIGHT = 1.0


def _prologue_body(cb_ref, sc_ref, iota_ref):
    c = cb_ref[...]
    sc_ref[...] = jnp.sum(c * c, axis=1).reshape(1, CB)
    iota_ref[...] = lax.broadcasted_iota(
        jnp.int32, (BLK, GW), 1).astype(jnp.float32)


def _tc_prologue(codebook):
    return pl.pallas_call(
        _prologue_body,
        out_shape=[
            jax.ShapeDtypeStruct((1, CB), jnp.float32),
            jax.ShapeDtypeStruct((BLK, GW), jnp.float32),
        ],
    )(codebook)


def _dist_argmin_body(z_ref, cb_ref, sc_ref, iota_ref, idx_ref, dsum_ref):
    step = pl.program_id(0)
    z = z_ref[...]
    c = cb_ref[...]
    sz = jnp.sum(z * z, axis=1, keepdims=True)
    z2 = 2.0 * z
    mm2 = lax.dot_general(
        z2, c,
        dimension_numbers=(((1,), (1,)), ((), ())),
        preferred_element_type=jnp.float32,
    )
    d = (sz + sc_ref[...]) - mm2
    vmin = None
    for g in range(NG):
        dg = d[:, g * GW:(g + 1) * GW]
        mg = jnp.min(dg, axis=1, keepdims=True)
        hit = dg == mg
        igf = jnp.min(jnp.where(hit, iota_ref[...], jnp.float32(GW)),
                      axis=1, keepdims=True)
        ig = igf.astype(jnp.int32) + g * GW
        bg = mg.astype(jnp.bfloat16).astype(jnp.float32)
        if g == 0:
            vmin, idx, carry_b = mg, ig, bg
        else:
            take = mg < carry_b
            idx = jnp.where(take, ig, idx)
            carry_b = jnp.where(take, bg, carry_b)
            vmin = jnp.minimum(vmin, mg)
    idx_ref[...] = idx[:, 0]

    @pl.when(step == 0)
    def _init():
        dsum_ref[0, 0] = 0.0

    dsum_ref[0, 0] += jnp.sum(vmin)


def _tc_dist_argmin(flat_z, codebook):
    sc_row, iota_f = _tc_prologue(codebook)
    return pl.pallas_call(
        _dist_argmin_body,
        grid=(GRID,),
        in_specs=[
            pl.BlockSpec((BLK, D), lambda i: (i, 0)),
            pl.BlockSpec((CB, D), lambda i: (0, 0)),
            pl.BlockSpec((1, CB), lambda i: (0, 0)),
            pl.BlockSpec((BLK, GW), lambda i: (0, 0)),
        ],
        out_specs=[
            pl.BlockSpec((BLK,), lambda i: (i,)),
            pl.BlockSpec(memory_space=pltpu.SMEM),
        ],
        out_shape=[
            jax.ShapeDtypeStruct((N_ROWS,), jnp.int32),
            jax.ShapeDtypeStruct((1, 1), jnp.float32),
        ],
    )(flat_z, codebook, sc_row, iota_f)


def _sc_gather_body(cb_hbm, idx_hbm, out_hbm, idx_v, rows_v, sem):
    wid = lax.axis_index("s") * NC + lax.axis_index("c")
    pltpu.sync_copy(idx_hbm.at[wid], idx_v)
    copies = [
        pltpu.async_copy(
            cb_hbm.at[idx_v.at[j]], rows_v.at[pl.ds(j * CH, CH)], sem
        )
        for j in range(NCH)
    ]
    for cp in copies:
        cp.wait()
    pltpu.sync_copy(rows_v, out_hbm.at[pl.ds(wid * BPW, BPW)])


@functools.cache
def _sc_gather():
    return pl.kernel(
        _sc_gather_body,
        mesh=plsc.VectorSubcoreMesh(core_axis_name="c", subcore_axis_name="s"),
        out_type=jax.ShapeDtypeStruct((N_ROWS, D), jnp.float32),
        scratch_types=[
            pltpu.VMEM((NCH, CH), jnp.int32),
            pltpu.VMEM((BPW, D), jnp.float32),
            pltpu.SemaphoreType.DMA,
        ],
        compiler_params=pltpu.CompilerParams(use_tc_tiling_on_sc=False),
    )


def kernel(z, codebook):
    b, l, h = z.shape
    flat_z = z.reshape(b * l, h)
    idx_flat, dsum = _tc_dist_argmin(flat_z, codebook)
    q = _sc_gather()(codebook, idx_flat.reshape(NW, NCH, CH))
    mse = dsum[0, 0] / jnp.float32(b * l * h)
    loss = _COMMIT_WEIGHT * mse + _QUANT_WEIGHT * mse
    return (
        q.reshape(b, l, h),
        idx_flat.reshape(b, l),
        loss,
        mse,
        mse,
    )


# speedup vs baseline: 10.6793x; 1.0025x over previous
import functools

import jax
import jax.numpy as jnp
from jax import lax
from jax.experimental import pallas as pl
from jax.experimental.pallas import tpu as pltpu
from jax.experimental.pallas import tpu_sc as plsc

CB = 8192
D = 64
N_ROWS = 9216
BLK = 256
GRID = N_ROWS // BLK
NG = 4
GW = CB // NG

NC = 2
NS = 16
NW = NC * NS
BPW = N_ROWS // NW
CH = 96
NCH = BPW // CH

_COMMIT_WEIGHT = 0.25
_QUANT_WEIGHT = 1.0


def _prologue_body(cb_ref, sc_ref, iota_ref):
    c = cb_ref[...]
    sc_ref[...] = jnp.sum(c * c, axis=1).reshape(1, CB)
    iota_ref[...] = lax.broadcasted_iota(
        jnp.int32, (BLK, GW), 1).astype(jnp.float32)


def _tc_prologue(codebook):
    return pl.pallas_call(
        _prologue_body,
        out_shape=[
            jax.ShapeDtypeStruct((1, CB), jnp.float32),
            jax.ShapeDtypeStruct((BLK, GW), jnp.float32),
        ],
    )(codebook)


def _dist_argmin_body(z_ref, cb_ref, sc_ref, iota_ref, idx_ref, dmin_ref):
    z = z_ref[...]
    c = cb_ref[...]
    sz = jnp.sum(z * z, axis=1, keepdims=True)
    z2 = 2.0 * z
    mm2 = lax.dot_general(
        z2, c,
        dimension_numbers=(((1,), (1,)), ((), ())),
        preferred_element_type=jnp.float32,
    )
    d = (sz + sc_ref[...]) - mm2
    vmin = None
    for g in range(NG):
        dg = d[:, g * GW:(g + 1) * GW]
        mg = jnp.min(dg, axis=1, keepdims=True)
        hit = dg == mg
        igf = jnp.min(jnp.where(hit, iota_ref[...], jnp.float32(GW)),
                      axis=1, keepdims=True)
        ig = igf.astype(jnp.int32) + g * GW
        bg = mg.astype(jnp.bfloat16).astype(jnp.float32)
        if g == 0:
            vmin, idx, carry_b = mg, ig, bg
        else:
            take = mg < carry_b
            idx = jnp.where(take, ig, idx)
            carry_b = jnp.where(take, bg, carry_b)
            vmin = jnp.minimum(vmin, mg)
    idx_ref[...] = idx[:, 0]
    dmin_ref[...] = vmin


def _tc_dist_argmin(flat_z, codebook):
    sc_row, iota_f = _tc_prologue(codebook)
    return pl.pallas_call(
        _dist_argmin_body,
        grid=(GRID,),
        in_specs=[
            pl.BlockSpec((BLK, D), lambda i: (i, 0)),
            pl.BlockSpec((CB, D), lambda i: (0, 0)),
            pl.BlockSpec((1, CB), lambda i: (0, 0)),
            pl.BlockSpec((BLK, GW), lambda i: (0, 0)),
        ],
        out_specs=[
            pl.BlockSpec((BLK,), lambda i: (i,)),
            pl.BlockSpec((BLK, 1), lambda i: (i, 0)),
        ],
        out_shape=[
            jax.ShapeDtypeStruct((N_ROWS,), jnp.int32),
            jax.ShapeDtypeStruct((N_ROWS, 1), jnp.float32),
        ],
        compiler_params=pltpu.CompilerParams(
            dimension_semantics=("parallel",),
        ),
    )(flat_z, codebook, sc_row, iota_f)


def _sc_gather_body(cb_hbm, idx_hbm, out_hbm, idx_v, rows_v, sem):
    wid = lax.axis_index("s") * NC + lax.axis_index("c")
    pltpu.sync_copy(idx_hbm.at[wid], idx_v)
    copies = [
        pltpu.async_copy(
            cb_hbm.at[idx_v.at[j]], rows_v.at[pl.ds(j * CH, CH)], sem
        )
        for j in range(NCH)
    ]
    for cp in copies:
        cp.wait()
    pltpu.sync_copy(rows_v, out_hbm.at[pl.ds(wid * BPW, BPW)])


@functools.cache
def _sc_gather():
    return pl.kernel(
        _sc_gather_body,
        mesh=plsc.VectorSubcoreMesh(core_axis_name="c", subcore_axis_name="s"),
        out_type=jax.ShapeDtypeStruct((N_ROWS, D), jnp.float32),
        scratch_types=[
            pltpu.VMEM((NCH, CH), jnp.int32),
            pltpu.VMEM((BPW, D), jnp.float32),
            pltpu.SemaphoreType.DMA,
        ],
        compiler_params=pltpu.CompilerParams(use_tc_tiling_on_sc=False),
    )


def kernel(z, codebook):
    b, l, h = z.shape
    flat_z = z.reshape(b * l, h)
    idx_flat, dmin = _tc_dist_argmin(flat_z, codebook)
    q = _sc_gather()(codebook, idx_flat.reshape(NW, NCH, CH))
    mse = jnp.sum(dmin) / jnp.float32(b * l * h)
    loss = _COMMIT_WEIGHT * mse + _QUANT_WEIGHT * mse
    return (
        q.reshape(b, l, h),
        idx_flat.reshape(b, l),
        loss,
        mse,
        mse,
    )


# speedup vs baseline: 10.7636x; 1.0079x over previous
import functools

import jax
import jax.numpy as jnp
from jax import lax
from jax.experimental import pallas as pl
from jax.experimental.pallas import tpu as pltpu
from jax.experimental.pallas import tpu_sc as plsc

CB = 8192
D = 64
N_ROWS = 9216
BLK = 512
GRID = N_ROWS // BLK
NG = 4
GW = CB // NG

NC = 2
NS = 16
NW = NC * NS
BPW = N_ROWS // NW
CH = 96
NCH = BPW // CH

_COMMIT_WEIGHT = 0.25
_QUANT_WEIGHT = 1.0


def _prologue_body(cb_ref, sc_ref, iota_ref):
    c = cb_ref[...]
    sc_ref[...] = jnp.sum(c * c, axis=1).reshape(1, CB)
    iota_ref[...] = lax.broadcasted_iota(
        jnp.int32, (BLK, GW), 1).astype(jnp.float32)


def _tc_prologue(codebook):
    return pl.pallas_call(
        _prologue_body,
        out_shape=[
            jax.ShapeDtypeStruct((1, CB), jnp.float32),
            jax.ShapeDtypeStruct((BLK, GW), jnp.float32),
        ],
    )(codebook)


def _dist_argmin_body(z_ref, cb_ref, sc_ref, iota_ref, idx_ref, dmin_ref):
    z = z_ref[...]
    c = cb_ref[...]
    sz = jnp.sum(z * z, axis=1, keepdims=True)
    z2 = 2.0 * z
    mm2 = lax.dot_general(
        z2, c,
        dimension_numbers=(((1,), (1,)), ((), ())),
        preferred_element_type=jnp.float32,
    )
    d = (sz + sc_ref[...]) - mm2
    vmin = None
    for g in range(NG):
        dg = d[:, g * GW:(g + 1) * GW]
        mg = jnp.min(dg, axis=1, keepdims=True)
        hit = dg == mg
        igf = jnp.min(jnp.where(hit, iota_ref[...], jnp.float32(GW)),
                      axis=1, keepdims=True)
        ig = igf.astype(jnp.int32) + g * GW
        bg = mg.astype(jnp.bfloat16).astype(jnp.float32)
        if g == 0:
            vmin, idx, carry_b = mg, ig, bg
        else:
            take = mg < carry_b
            idx = jnp.where(take, ig, idx)
            carry_b = jnp.where(take, bg, carry_b)
            vmin = jnp.minimum(vmin, mg)
    idx_ref[...] = idx[:, 0]
    dmin_ref[...] = vmin


def _tc_dist_argmin(flat_z, codebook):
    sc_row, iota_f = _tc_prologue(codebook)
    return pl.pallas_call(
        _dist_argmin_body,
        grid=(GRID,),
        in_specs=[
            pl.BlockSpec((BLK, D), lambda i: (i, 0)),
            pl.BlockSpec((CB, D), lambda i: (0, 0)),
            pl.BlockSpec((1, CB), lambda i: (0, 0)),
            pl.BlockSpec((BLK, GW), lambda i: (0, 0)),
        ],
        out_specs=[
            pl.BlockSpec((BLK,), lambda i: (i,)),
            pl.BlockSpec((BLK, 1), lambda i: (i, 0)),
        ],
        out_shape=[
            jax.ShapeDtypeStruct((N_ROWS,), jnp.int32),
            jax.ShapeDtypeStruct((N_ROWS, 1), jnp.float32),
        ],
        compiler_params=pltpu.CompilerParams(
            dimension_semantics=("parallel",),
        ),
    )(flat_z, codebook, sc_row, iota_f)


def _sc_gather_body(cb_hbm, idx_hbm, out_hbm, idx_v, rows_v, sem):
    wid = lax.axis_index("s") * NC + lax.axis_index("c")
    pltpu.sync_copy(idx_hbm.at[wid], idx_v)
    copies = [
        pltpu.async_copy(
            cb_hbm.at[idx_v.at[j]], rows_v.at[pl.ds(j * CH, CH)], sem
        )
        for j in range(NCH)
    ]
    for cp in copies:
        cp.wait()
    pltpu.sync_copy(rows_v, out_hbm.at[pl.ds(wid * BPW, BPW)])


@functools.cache
def _sc_gather():
    return pl.kernel(
        _sc_gather_body,
        mesh=plsc.VectorSubcoreMesh(core_axis_name="c", subcore_axis_name="s"),
        out_type=jax.ShapeDtypeStruct((N_ROWS, D), jnp.float32),
        scratch_types=[
            pltpu.VMEM((NCH, CH), jnp.int32),
            pltpu.VMEM((BPW, D), jnp.float32),
            pltpu.SemaphoreType.DMA,
        ],
        compiler_params=pltpu.CompilerParams(use_tc_tiling_on_sc=False),
    )


def kernel(z, codebook):
    b, l, h = z.shape
    flat_z = z.reshape(b * l, h)
    idx_flat, dmin = _tc_dist_argmin(flat_z, codebook)
    q = _sc_gather()(codebook, idx_flat.reshape(NW, NCH, CH))
    mse = jnp.sum(dmin) / jnp.float32(b * l * h)
    loss = _COMMIT_WEIGHT * mse + _QUANT_WEIGHT * mse
    return (
        q.reshape(b, l, h),
        idx_flat.reshape(b, l),
        loss,
        mse,
        mse,
    )


# speedup vs baseline: 10.8483x; 1.0079x over previous
import functools

import jax
import jax.numpy as jnp
from jax import lax
from jax.experimental import pallas as pl
from jax.experimental.pallas import tpu as pltpu
from jax.experimental.pallas import tpu_sc as plsc

CB = 8192
D = 64
N_ROWS = 9216
BLK = 512
GRID = N_ROWS // BLK
NG = 4
GW = CB // NG

NC = 2
NS = 16
NW = NC * NS
BPW = N_ROWS // NW
CH = 96
NCH = BPW // CH

_COMMIT_WEIGHT = 0.25
_QUANT_WEIGHT = 1.0


def _prologue_body(cb_ref, sc_ref, iota_ref):
    c = cb_ref[...]
    sc_ref[...] = jnp.sum(c * c, axis=1).reshape(1, CB)
    iota_ref[...] = lax.broadcasted_iota(
        jnp.int32, (BLK, GW), 1).astype(jnp.float32)


def _tc_prologue(codebook):
    return pl.pallas_call(
        _prologue_body,
        out_shape=[
            jax.ShapeDtypeStruct((1, CB), jnp.float32),
            jax.ShapeDtypeStruct((BLK, GW), jnp.float32),
        ],
    )(codebook)


def _dist_argmin_body(z_ref, cb_ref, sc_ref, iota_ref, idx_ref, dmin_ref):
    z = z_ref[...]
    c = cb_ref[...]
    sz = jnp.sum(z * z, axis=1, keepdims=True)
    z2 = 2.0 * z
    mm2 = lax.dot_general(
        z2, c,
        dimension_numbers=(((1,), (1,)), ((), ())),
        preferred_element_type=jnp.float32,
    )
    d = (sz + sc_ref[...]) - mm2
    vmin = None
    for g in range(NG):
        dg = d[:, g * GW:(g + 1) * GW]
        mg = jnp.min(dg, axis=1, keepdims=True)
        hit = dg == mg
        iota_f = lax.broadcasted_iota(
            jnp.int32, (BLK, GW), 1).astype(jnp.float32)
        igf = jnp.min(jnp.where(hit, iota_f, jnp.float32(GW)),
                      axis=1, keepdims=True)
        ig = igf.astype(jnp.int32) + g * GW
        bg = mg.astype(jnp.bfloat16).astype(jnp.float32)
        if g == 0:
            vmin, idx, carry_b = mg, ig, bg
        else:
            take = mg < carry_b
            idx = jnp.where(take, ig, idx)
            carry_b = jnp.where(take, bg, carry_b)
            vmin = jnp.minimum(vmin, mg)
    idx_ref[...] = idx[:, 0]
    dmin_ref[...] = vmin


def _tc_dist_argmin(flat_z, codebook):
    sc_row, iota_f = _tc_prologue(codebook)
    return pl.pallas_call(
        _dist_argmin_body,
        grid=(GRID,),
        in_specs=[
            pl.BlockSpec((BLK, D), lambda i: (i, 0)),
            pl.BlockSpec((CB, D), lambda i: (0, 0)),
            pl.BlockSpec((1, CB), lambda i: (0, 0)),
            pl.BlockSpec((BLK, GW), lambda i: (0, 0)),
        ],
        out_specs=[
            pl.BlockSpec((BLK,), lambda i: (i,)),
            pl.BlockSpec((BLK, 1), lambda i: (i, 0)),
        ],
        out_shape=[
            jax.ShapeDtypeStruct((N_ROWS,), jnp.int32),
            jax.ShapeDtypeStruct((N_ROWS, 1), jnp.float32),
        ],
        compiler_params=pltpu.CompilerParams(
            dimension_semantics=("parallel",),
        ),
    )(flat_z, codebook, sc_row, iota_f)


def _sc_gather_body(cb_hbm, idx_hbm, out_hbm, idx_v, rows_v, sem):
    wid = lax.axis_index("s") * NC + lax.axis_index("c")
    pltpu.sync_copy(idx_hbm.at[wid], idx_v)
    copies = [
        pltpu.async_copy(
            cb_hbm.at[idx_v.at[j]], rows_v.at[pl.ds(j * CH, CH)], sem
        )
        for j in range(NCH)
    ]
    for cp in copies:
        cp.wait()
    pltpu.sync_copy(rows_v, out_hbm.at[pl.ds(wid * BPW, BPW)])


@functools.cache
def _sc_gather():
    return pl.kernel(
        _sc_gather_body,
        mesh=plsc.VectorSubcoreMesh(core_axis_name="c", subcore_axis_name="s"),
        out_type=jax.ShapeDtypeStruct((N_ROWS, D), jnp.float32),
        scratch_types=[
            pltpu.VMEM((NCH, CH), jnp.int32),
            pltpu.VMEM((BPW, D), jnp.float32),
            pltpu.SemaphoreType.DMA,
        ],
        compiler_params=pltpu.CompilerParams(use_tc_tiling_on_sc=False),
    )


def kernel(z, codebook):
    b, l, h = z.shape
    flat_z = z.reshape(b * l, h)
    idx_flat, dmin = _tc_dist_argmin(flat_z, codebook)
    q = _sc_gather()(codebook, idx_flat.reshape(NW, NCH, CH))
    mse = jnp.sum(dmin) / jnp.float32(b * l * h)
    loss = _COMMIT_WEIGHT * mse + _QUANT_WEIGHT * mse
    return (
        q.reshape(b, l, h),
        idx_flat.reshape(b, l),
        loss,
        mse,
        mse,
    )


# speedup vs baseline: 11.0403x; 1.0177x over previous
import functools

import jax
import jax.numpy as jnp
from jax import lax
from jax.experimental import pallas as pl
from jax.experimental.pallas import tpu as pltpu
from jax.experimental.pallas import tpu_sc as plsc

CB = 8192
D = 64
N_ROWS = 9216
BLK = 512
GRID = N_ROWS // BLK
NG = 4
GW = CB // NG

NC = 2
NS = 16
NW = NC * NS
BPW = N_ROWS // NW
CH = 96
NCH = BPW // CH

_COMMIT_WEIGHT = 0.25
_QUANT_WEIGHT = 1.0


def _prologue_body(cb_ref, sc_ref):
    c = cb_ref[...]
    sc_ref[...] = jnp.sum(c * c, axis=1).reshape(1, CB)


def _tc_prologue(codebook):
    return pl.pallas_call(
        _prologue_body,
        out_shape=jax.ShapeDtypeStruct((1, CB), jnp.float32),
    )(codebook)


def _dist_argmin_body(z_ref, cb_ref, sc_ref, idx_ref, dmin_ref):
    z = z_ref[...]
    c = cb_ref[...]
    sz = jnp.sum(z * z, axis=1, keepdims=True)
    z2 = 2.0 * z
    mm2 = lax.dot_general(
        z2, c,
        dimension_numbers=(((1,), (1,)), ((), ())),
        preferred_element_type=jnp.float32,
    )
    d = (sz + sc_ref[...]) - mm2
    vmin = None
    for g in range(NG):
        dg = d[:, g * GW:(g + 1) * GW]
        mg = jnp.min(dg, axis=1, keepdims=True)
        hit = dg == mg
        iota_f = lax.broadcasted_iota(
            jnp.int32, (BLK, GW), 1).astype(jnp.float32)
        igf = jnp.min(jnp.where(hit, iota_f, jnp.float32(GW)),
                      axis=1, keepdims=True)
        ig = igf.astype(jnp.int32) + g * GW
        bg = mg.astype(jnp.bfloat16).astype(jnp.float32)
        if g == 0:
            vmin, idx, carry_b = mg, ig, bg
        else:
            take = mg < carry_b
            idx = jnp.where(take, ig, idx)
            carry_b = jnp.where(take, bg, carry_b)
            vmin = jnp.minimum(vmin, mg)
    idx_ref[...] = idx[:, 0]
    dmin_ref[...] = vmin


def _tc_dist_argmin(flat_z, codebook):
    sc_row = _tc_prologue(codebook)
    return pl.pallas_call(
        _dist_argmin_body,
        grid=(GRID,),
        in_specs=[
            pl.BlockSpec((BLK, D), lambda i: (i, 0)),
            pl.BlockSpec((CB, D), lambda i: (0, 0)),
            pl.BlockSpec((1, CB), lambda i: (0, 0)),
        ],
        out_specs=[
            pl.BlockSpec((BLK,), lambda i: (i,)),
            pl.BlockSpec((BLK, 1), lambda i: (i, 0)),
        ],
        out_shape=[
            jax.ShapeDtypeStruct((N_ROWS,), jnp.int32),
            jax.ShapeDtypeStruct((N_ROWS, 1), jnp.float32),
        ],
        compiler_params=pltpu.CompilerParams(
            dimension_semantics=("parallel",),
        ),
    )(flat_z, codebook, sc_row)


def _sc_gather_body(cb_hbm, idx_hbm, out_hbm, idx_v, rows_v, sem):
    wid = lax.axis_index("s") * NC + lax.axis_index("c")
    pltpu.sync_copy(idx_hbm.at[wid], idx_v)
    copies = [
        pltpu.async_copy(
            cb_hbm.at[idx_v.at[j]], rows_v.at[pl.ds(j * CH, CH)], sem
        )
        for j in range(NCH)
    ]
    for cp in copies:
        cp.wait()
    pltpu.sync_copy(rows_v, out_hbm.at[pl.ds(wid * BPW, BPW)])


@functools.cache
def _sc_gather():
    return pl.kernel(
        _sc_gather_body,
        mesh=plsc.VectorSubcoreMesh(core_axis_name="c", subcore_axis_name="s"),
        out_type=jax.ShapeDtypeStruct((N_ROWS, D), jnp.float32),
        scratch_types=[
            pltpu.VMEM((NCH, CH), jnp.int32),
            pltpu.VMEM((BPW, D), jnp.float32),
            pltpu.SemaphoreType.DMA,
        ],
        compiler_params=pltpu.CompilerParams(use_tc_tiling_on_sc=False),
    )


def kernel(z, codebook):
    b, l, h = z.shape
    flat_z = z.reshape(b * l, h)
    idx_flat, dmin = _tc_dist_argmin(flat_z, codebook)
    q = _sc_gather()(codebook, idx_flat.reshape(NW, NCH, CH))
    mse = jnp.sum(dmin) / jnp.float32(b * l * h)
    loss = _COMMIT_WEIGHT * mse + _QUANT_WEIGHT * mse
    return (
        q.reshape(b, l, h),
        idx_flat.reshape(b, l),
        loss,
        mse,
        mse,
    )
